# baseline (speedup 1.0000x reference)
import jax
import jax.numpy as jnp
from jax import lax
from jax.experimental import pallas as pl
from jax.experimental.pallas import tpu as pltpu
from jax.experimental.pallas import tpu_sc as plsc

M = 100000
D = 128
B = 16384
NC = 2
NS = 16
NW = NC * NS
G = 8
FCH = 160
NFC = M // FCH
FCPW = (NFC + NW - 1) // NW
NG2 = FCH // G
LW = 25
LS = M // LW
LSTEPS = LS // 16


def _body(mf_hbm, ml_hbm, bf_hbm, bl_hbm, ptr_hbm,
          outf_hbm, outl_hbm, lab_v, bl_v, ptr_v):
    cid = lax.axis_index("c")
    sid = lax.axis_index("s")
    wid = cid * NS + sid

    pltpu.sync_copy(ptr_hbm, ptr_v)
    p = ptr_v[...][0]

    def jmod(x):
        t = x - p
        return jnp.where(t < 0, t + M, t)

    def copy_rows(j_src, g_dst, n, from_batch):
        @pl.when(from_batch)
        def _():
            pltpu.sync_copy(bf_hbm.at[pl.ds(j_src, n)],
                            outf_hbm.at[pl.ds(g_dst, n)])

        @pl.when(jnp.logical_not(from_batch))
        def _():
            pltpu.sync_copy(mf_hbm.at[pl.ds(g_dst, n)],
                            outf_hbm.at[pl.ds(g_dst, n)])

    def cbody(i, carry):
        c = wid + i * NW

        @pl.when(c < NFC)
        def _():
            g0 = c * FCH
            j0 = jmod(g0)
            jl = jmod(g0 + FCH - 1)
            jump = jnp.logical_and(p > g0, p < g0 + FCH)
            inw0 = j0 < B
            inwl = jl < B
            clean = jnp.logical_and(jnp.logical_not(jump), inw0 == inwl)

            @pl.when(clean)
            def _():
                copy_rows(j0, g0, FCH, inw0)

            @pl.when(jnp.logical_not(clean))
            def _():
                def gbody(gi, carry2):
                    gg = g0 + gi * G
                    jg = jmod(gg)
                    jgl = jmod(gg + G - 1)
                    gjump = jnp.logical_and(p > gg, p < gg + G)
                    gin0 = jg < B
                    ginl = jgl < B
                    gclean = jnp.logical_and(jnp.logical_not(gjump),
                                             gin0 == ginl)

                    @pl.when(gclean)
                    def _():
                        copy_rows(jg, gg, G, gin0)

                    @pl.when(jnp.logical_not(gclean))
                    def _():
                        for r in range(G):
                            jr = jmod(gg + r)
                            copy_rows(jr, gg + r, 1, jr < B)

                    return carry2
                lax.fori_loop(0, NG2, gbody, 0)

        return carry
    lax.fori_loop(0, FCPW, cbody, 0)

    @pl.when(wid < LW)
    def _():
        s0 = wid * LS
        pltpu.sync_copy(ml_hbm.at[pl.ds(s0, LS)], lab_v)
        pltpu.sync_copy(bl_hbm, bl_v)
        lanes = lax.iota(jnp.int32, 16)

        def lbody(i, carry):
            off = i * 16
            g = s0 + off + lanes
            t1 = g - p
            j = jnp.where(t1 < 0, t1 + M, t1)
            mask = j < B
            jc = jnp.where(mask, j, 0)
            gathered = plsc.load_gather(bl_v, [jc])
            cur = lab_v[pl.ds(off, 16)]
            lab_v[pl.ds(off, 16)] = jnp.where(mask, gathered, cur)
            return carry
        lax.fori_loop(0, LSTEPS, lbody, 0)
        pltpu.sync_copy(lab_v, outl_hbm.at[pl.ds(s0, LS)])


_rewrite = pl.kernel(
    _body,
    out_type=(jax.ShapeDtypeStruct((M, D), jnp.float32),
              jax.ShapeDtypeStruct((M,), jnp.int32)),
    mesh=plsc.VectorSubcoreMesh(core_axis_name="c", subcore_axis_name="s",
                                num_cores=NC, num_subcores=NS),
    compiler_params=pltpu.CompilerParams(use_tc_tiling_on_sc=False,
                                         needs_layout_passes=False),
    scratch_types=[
        pltpu.VMEM((LS,), jnp.int32),
        pltpu.VMEM((B,), jnp.int32),
        pltpu.VMEM((16,), jnp.int32),
    ],
)


def kernel(memory_features, memory_labels, batch_features, batch_labels, ptr):
    ptr32 = jnp.asarray(ptr, jnp.int32)
    ptr_arr = jnp.full((16,), ptr32, dtype=jnp.int32)
    new_features, new_labels = _rewrite(memory_features, memory_labels,
                                        batch_features, batch_labels, ptr_arr)
    new_ptr = (ptr32 + B) % M
    return new_features, new_labels, new_ptr


# speedup vs baseline: 25.2012x; 25.2012x over previous
import jax
import jax.numpy as jnp
from jax import lax
from jax.experimental import pallas as pl
from jax.experimental.pallas import tpu as pltpu
from jax.experimental.pallas import tpu_sc as plsc

M = 100000
D = 128
B = 16384
NC = 2
NS = 16
NW = NC * NS
G = 8
FCH = 160
NFC = M // FCH
FCPW = (NFC + NW - 1) // NW
NG2 = FCH // G
NB = 4
NGRP = FCPW // NB
LW = 25
LS = M // LW
LSTEPS = LS // 16


def _body(mf_hbm, ml_hbm, bf_hbm, bl_hbm, ptr_hbm,
          outf_hbm, outl_hbm,
          fb0, fb1, fb2, fb3, lab_v, bl_v, ptr_v,
          is0, is1, is2, is3, os0, os1, os2, os3, lsem0, lsem1):
    fbufs = (fb0, fb1, fb2, fb3)
    in_sems = (is0, is1, is2, is3)
    out_sems = (os0, os1, os2, os3)

    cid = lax.axis_index("c")
    sid = lax.axis_index("s")
    wid = cid * NS + sid

    pltpu.sync_copy(ptr_hbm, ptr_v)
    p = ptr_v[...][0]

    s0 = wid * LS
    is_lab = wid < LW

    @pl.when(is_lab)
    def _():
        pltpu.make_async_copy(ml_hbm.at[pl.ds(s0, LS)], lab_v, lsem0).start()
        pltpu.make_async_copy(bl_hbm, bl_v, lsem1).start()

    def jmod(x):
        t = x - p
        return jnp.where(t < 0, t + M, t)

    def classify(g0, n):
        j0 = jmod(g0)
        jl = jmod(g0 + n - 1)
        jump = jnp.logical_and(p > g0, p < g0 + n)
        inw0 = j0 < B
        clean = jnp.logical_and(jnp.logical_not(jump), inw0 == (jl < B))
        return j0, clean, inw0

    def chunk_of(gq, b):
        return wid + (gq * NB + b) * NW

    def start_in(gq, b):
        c = chunk_of(gq, b)

        @pl.when(c < NFC)
        def _():
            g0 = c * FCH
            j0, clean, inw0 = classify(g0, FCH)
            from_batch = jnp.logical_and(clean, inw0)

            @pl.when(from_batch)
            def _():
                pltpu.make_async_copy(bf_hbm.at[pl.ds(j0, FCH)],
                                      fbufs[b], in_sems[b]).start()

            @pl.when(jnp.logical_not(from_batch))
            def _():
                pltpu.make_async_copy(mf_hbm.at[pl.ds(g0, FCH)],
                                      fbufs[b], in_sems[b]).start()

    def finish_chunk(gq, b):
        c = chunk_of(gq, b)

        @pl.when(c < NFC)
        def _():
            g0 = c * FCH
            _, clean, _ = classify(g0, FCH)
            pltpu.make_async_copy(mf_hbm.at[pl.ds(0, FCH)],
                                  fbufs[b], in_sems[b]).wait()

            @pl.when(jnp.logical_not(clean))
            def _():
                def gbody(gi, carry2):
                    gg = g0 + gi * G
                    jg, gclean, gin0 = classify(gg, G)

                    @pl.when(jnp.logical_and(gclean, gin0))
                    def _():
                        pltpu.sync_copy(bf_hbm.at[pl.ds(jg, G)],
                                        fbufs[b].at[pl.ds(gi * G, G)])

                    @pl.when(jnp.logical_not(gclean))
                    def _():
                        for r in range(G):
                            jr = jmod(gg + r)

                            @pl.when(jr < B)
                            def _():
                                pltpu.sync_copy(
                                    bf_hbm.at[pl.ds(jr, 1)],
                                    fbufs[b].at[pl.ds(gi * G + r, 1)])

                    return carry2
                lax.fori_loop(0, NG2, gbody, 0)

            pltpu.make_async_copy(fbufs[b], outf_hbm.at[pl.ds(g0, FCH)],
                                  out_sems[b]).start()

    def drain_out(gq, b):
        c = chunk_of(gq, b)

        @pl.when(c < NFC)
        def _():
            pltpu.make_async_copy(fbufs[b], outf_hbm.at[pl.ds(0, FCH)],
                                  out_sems[b]).wait()

    def group(gq, carry):
        for b in range(NB):
            @pl.when(gq > 0)
            def _():
                drain_out(gq - 1, b)
            start_in(gq, b)
        for b in range(NB):
            finish_chunk(gq, b)
        return carry
    lax.fori_loop(0, NGRP, group, 0)

    @pl.when(is_lab)
    def _():
        pltpu.make_async_copy(ml_hbm.at[pl.ds(0, LS)], lab_v, lsem0).wait()
        pltpu.make_async_copy(bl_hbm, bl_v, lsem1).wait()
        lanes = lax.iota(jnp.int32, 16)

        def lbody(i, carry):
            off = i * 16
            g = s0 + off + lanes
            t1 = g - p
            j = jnp.where(t1 < 0, t1 + M, t1)
            mask = j < B
            jc = jnp.where(mask, j, 0)
            gathered = plsc.load_gather(bl_v, [jc])
            cur = lab_v[pl.ds(off, 16)]
            lab_v[pl.ds(off, 16)] = jnp.where(mask, gathered, cur)
            return carry
        lax.fori_loop(0, LSTEPS, lbody, 0)
        pltpu.sync_copy(lab_v, outl_hbm.at[pl.ds(s0, LS)])

    for b in range(NB):
        drain_out(NGRP - 1, b)


_rewrite = pl.kernel(
    _body,
    out_type=(jax.ShapeDtypeStruct((M, D), jnp.float32),
              jax.ShapeDtypeStruct((M,), jnp.int32)),
    mesh=plsc.VectorSubcoreMesh(core_axis_name="c", subcore_axis_name="s",
                                num_cores=NC, num_subcores=NS),
    compiler_params=pltpu.CompilerParams(use_tc_tiling_on_sc=False,
                                         needs_layout_passes=False),
    scratch_types=[
        pltpu.VMEM((FCH, D), jnp.float32),
        pltpu.VMEM((FCH, D), jnp.float32),
        pltpu.VMEM((FCH, D), jnp.float32),
        pltpu.VMEM((FCH, D), jnp.float32),
        pltpu.VMEM((LS,), jnp.int32),
        pltpu.VMEM((B,), jnp.int32),
        pltpu.VMEM((16,), jnp.int32),
        pltpu.SemaphoreType.DMA,
        pltpu.SemaphoreType.DMA,
        pltpu.SemaphoreType.DMA,
        pltpu.SemaphoreType.DMA,
        pltpu.SemaphoreType.DMA,
        pltpu.SemaphoreType.DMA,
        pltpu.SemaphoreType.DMA,
        pltpu.SemaphoreType.DMA,
        pltpu.SemaphoreType.DMA,
        pltpu.SemaphoreType.DMA,
    ],
)


def kernel(memory_features, memory_labels, batch_features, batch_labels, ptr):
    ptr32 = jnp.asarray(ptr, jnp.int32)
    ptr_arr = jnp.full((16,), ptr32, dtype=jnp.int32)
    new_features, new_labels = _rewrite(memory_features, memory_labels,
                                        batch_features, batch_labels, ptr_arr)
    new_ptr = (ptr32 + B) % M
    return new_features, new_labels, new_ptr
